# Initial kernel scaffold; baseline (speedup 1.0000x reference)
import jax
import jax.numpy as jnp
from jax.experimental import pallas as pl


def kernel(pos):
    raise NotImplementedError("write your pallas kernel here")


# speedup vs baseline: 28.8810x; 28.8810x over previous
import jax
import jax.numpy as jnp
from jax import lax
from jax.experimental import pallas as pl
from jax.experimental.pallas import tpu as pltpu
from jax.experimental.pallas import tpu_sc as plsc

_B = 16
_N = 8192
_M = 1024
_L = 16
_H = _N // 2
_NCHUNK = _H // _L
_SLOT = 2 * _L


def _fps_body(pos_hbm, out_hbm, px, py, pz, dists, ox, oy, oz, xbuf, sh):
    c = lax.axis_index("c")
    s = lax.axis_index("s")
    b = c * 8 + lax.div(s, 2)
    h = lax.rem(s, 2)
    base = h * _H

    pltpu.sync_copy(pos_hbm.at[pl.ds((b * 3 + 0) * _N, _N)], px)
    pltpu.sync_copy(pos_hbm.at[pl.ds((b * 3 + 1) * _N, _N)], py)
    pltpu.sync_copy(pos_hbm.at[pl.ds((b * 3 + 2) * _N, _N)], pz)

    lanes = lax.iota(jnp.int32, _L)
    lane0 = lanes == 0
    big = jnp.full((_L,), 3.4e38, jnp.float32)
    neg = jnp.full((_L,), -3.4e38, jnp.float32)
    nsent = jnp.full((_L,), _N, jnp.int32)

    @plsc.parallel_loop(0, _NCHUNK, unroll=8)
    def _init(k):
        dists[pl.ds(k * _L, _L)] = big

    sx = jnp.full((_L,), jnp.max(jnp.where(lane0, px[pl.ds(0, _L)], neg)),
                  jnp.float32)
    sy = jnp.full((_L,), jnp.max(jnp.where(lane0, py[pl.ds(0, _L)], neg)),
                  jnp.float32)
    sz = jnp.full((_L,), jnp.max(jnp.where(lane0, pz[pl.ds(0, _L)], neg)),
                  jnp.float32)

    @pl.when(h == 0)
    def _():
        zf = jnp.zeros((_L,), jnp.float32)
        ox[pl.ds(0, _L)] = jnp.where(lane0, sx, zf)
        oy[pl.ds(0, _L)] = jnp.where(lane0, sy, zf)
        oz[pl.ds(0, _L)] = jnp.where(lane0, sz, zf)

    def step(i, carry):
        cx, cy, cz = carry

        @plsc.parallel_loop(
            0, _NCHUNK, unroll=8,
            carry=(jnp.full((_L,), -1.0, jnp.float32),
                   jnp.zeros((_L,), jnp.int32)))
        def chunk(k, acc):
            best, bidx = acc
            off = pl.multiple_of(k * _L, _L)
            vx = px[pl.ds(base + off, _L)]
            vy = py[pl.ds(base + off, _L)]
            vz = pz[pl.ds(base + off, _L)]
            dx = vx - cx
            dy = vy - cy
            dz = vz - cz
            d = dx * dx + dy * dy + dz * dz
            nd = jnp.minimum(dists[pl.ds(off, _L)], d)
            dists[pl.ds(off, _L)] = nd
            m = nd > best
            best = jnp.where(m, nd, best)
            bidx = jnp.where(m, base + off + lanes, bidx)
            return best, bidx

        best, bidx = chunk
        mval = jnp.max(best)
        cand = jnp.where(best == mval, bidx, nsent)
        midx = jnp.min(cand)
        mv = jnp.full((_L,), mval, jnp.float32)
        mi = jnp.full((_L,), midx, jnp.int32)

        par = lax.rem(i, 2)
        my_slot = (s + 16 * par) * _SLOT
        pa_slot = ((s ^ 1) + 16 * par) * _SLOT
        xbuf[pl.ds(0, _L)] = plsc.bitcast(mv, jnp.int32)
        xbuf[pl.ds(_L, _L)] = mi
        pltpu.sync_copy(xbuf, sh.at[pl.ds(my_slot, _SLOT)])
        plsc.subcore_barrier()
        pltpu.sync_copy(sh.at[pl.ds(pa_slot, _SLOT)], xbuf)
        pv = plsc.bitcast(xbuf[pl.ds(0, _L)], jnp.float32)
        pi = xbuf[pl.ds(_L, _L)]
        win = (pv > mv) | ((pv == mv) & (pi < mi))
        wi = jnp.where(win, pi, mi)

        nsx = plsc.load_gather(px, [wi])
        nsy = plsc.load_gather(py, [wi])
        nsz = plsc.load_gather(pz, [wi])

        @pl.when(h == 0)
        def _():
            oi = jnp.full((_L,), i, jnp.int32)
            plsc.store_scatter(ox, [oi], nsx, mask=lane0)
            plsc.store_scatter(oy, [oi], nsy, mask=lane0)
            plsc.store_scatter(oz, [oi], nsz, mask=lane0)

        return nsx, nsy, nsz

    lax.fori_loop(1, _M, step, (sx, sy, sz))

    @pl.when(h == 0)
    def _():
        pltpu.sync_copy(ox, out_hbm.at[pl.ds((b * 3 + 0) * _M, _M)])
        pltpu.sync_copy(oy, out_hbm.at[pl.ds((b * 3 + 1) * _M, _M)])
        pltpu.sync_copy(oz, out_hbm.at[pl.ds((b * 3 + 2) * _M, _M)])


@jax.jit
def _fps(pos_t):
    mesh = plsc.VectorSubcoreMesh(core_axis_name="c", subcore_axis_name="s")
    k = pl.kernel(
        _fps_body,
        out_type=jax.ShapeDtypeStruct((_B * 3 * _M,), jnp.float32),
        mesh=mesh,
        scratch_types=[
            pltpu.VMEM((_N,), jnp.float32),
            pltpu.VMEM((_N,), jnp.float32),
            pltpu.VMEM((_N,), jnp.float32),
            pltpu.VMEM((_H,), jnp.float32),
            pltpu.VMEM((_M,), jnp.float32),
            pltpu.VMEM((_M,), jnp.float32),
            pltpu.VMEM((_M,), jnp.float32),
            pltpu.VMEM((_SLOT,), jnp.int32),
            pltpu.VMEM_SHARED((32 * _SLOT,), jnp.int32),
        ],
        compiler_params=pltpu.CompilerParams(needs_layout_passes=False),
    )
    return k(pos_t)


def kernel(pos):
    pos_t = pos.reshape(_B, _N, 3).transpose(0, 2, 1).reshape(-1)
    out = _fps(pos_t)
    return out.reshape(_B, 3, _M).transpose(0, 2, 1).reshape(_B * _M, 3)
